# baseline (speedup 1.0000x reference)
import functools

import jax
import jax.numpy as jnp
import numpy as np
from jax import lax
from jax.experimental import pallas as pl
from jax.experimental.pallas import tpu as pltpu
from jax.experimental.pallas import tpu_sc as plsc

U32 = jnp.uint32
I32 = jnp.int32
F32 = jnp.float32
_SCALE15625 = np.float32((1.0 - 2.0**-21) / 15625.0)


def _c(v):
    return U32(v)


def _mul64c(lo, hi, c):
    c = _c(c)
    l0 = lo & _c(0xFFFF)
    l1 = lo >> _c(16)
    p0 = l0 * c
    p1 = l1 * c
    nlo = (p1 << _c(16)) + p0
    ch = (p1 + (p0 >> _c(16))) >> _c(16)
    h0 = hi & _c(0xFFFF)
    h1 = hi >> _c(16)
    nhi = ((h1 * c) << _c(16)) + h0 * c + ch
    return nlo, nhi


def _xor_shr(lo, hi, s):
    slo = (lo >> _c(s)) | (hi << _c(32 - s))
    shi = hi >> _c(s)
    return lo ^ slo, hi ^ shi


def _hash_i32(v):
    return plsc.bitcast(_hash_vec(plsc.bitcast(v, U32)), I32)


def _hash_vec(x):
    blo = x << _c(21)
    bhi = x >> _c(11)
    alo = ~x
    lo = alo + blo
    carry = jnp.where(lo < alo, _c(1), _c(0))
    hi = bhi + carry + _c(0xFFFFFFFF)
    lo, hi = _xor_shr(lo, hi, 24)
    lo, hi = _mul64c(lo, hi, 265)
    lo, hi = _xor_shr(lo, hi, 14)
    lo, hi = _mul64c(lo, hi, 21)
    lo, hi = _xor_shr(lo, hi, 28)
    b0m = lo << _c(31)
    nlo = lo ^ b0m
    c7 = (lo & b0m) >> _c(31)
    shi = (hi << _c(31)) | (lo >> _c(1))
    hi = hi + shi + c7
    lo = nlo
    r0 = lo & _c(63)
    qlo = (lo >> _c(6)) | (hi << _c(26))
    qhi = hi >> _c(6)
    w0 = qlo & _c(0xFFFF)
    w1 = qlo >> _c(16)
    w2 = qhi & _c(0xFFFF)
    w3 = qhi >> _c(16)
    s = w0 + w1 * _c(3036) + w2 * _c(14171) + w3 * _c(7531)
    s = s.astype(I32)
    q = (s.astype(F32) * _SCALE15625).astype(I32)
    r = s - q * I32(15625)
    t = r - I32(15625)
    r = t + ((t >> I32(31)) & I32(15625))
    return (r.astype(U32) << _c(6)) | r0


def _make_sc_call(n):
    info = plsc.get_sparse_core_info()
    nc, ns = info.num_cores, info.num_subcores
    nw = nc * ns
    per_w = n // nw
    groups = 4
    chunk = per_w // groups
    assert per_w * nw == n and chunk * groups == per_w and chunk % 16 == 0
    mesh = plsc.VectorSubcoreMesh(core_axis_name="c", subcore_axis_name="s")

    @functools.partial(
        pl.kernel,
        mesh=mesh,
        out_type=jax.ShapeDtypeStruct((n,), jnp.int32),
        scratch_types=[pltpu.VMEM((per_w,), jnp.int32)],
    )
    def sc_hash(x_hbm, out_hbm, buf):
        wid = lax.axis_index("s") * I32(nc) + lax.axis_index("c")
        base = pl.multiple_of(wid * I32(per_w), per_w)
        pltpu.sync_copy(x_hbm.at[pl.ds(base, per_w)], buf)

        @plsc.parallel_loop(I32(0), I32(per_w), step=I32(16), unroll=8)
        def _(i):
            off = pl.multiple_of(i, 16)
            buf[pl.ds(off, 16)] = _hash_i32(buf[pl.ds(off, 16)])

        pltpu.sync_copy(buf, out_hbm.at[pl.ds(base, per_w)])

    return sc_hash


@jax.jit
def kernel(raw_ids):
    shape = raw_ids.shape
    n = raw_ids.size
    x = raw_ids.astype(jnp.int32).reshape(n)
    out = _make_sc_call(n)(x)
    return out.reshape(shape).astype(jnp.int64)


# speedup vs baseline: 1.0756x; 1.0756x over previous
import functools

import jax
import jax.numpy as jnp
import numpy as np
from jax import lax
from jax.experimental import pallas as pl
from jax.experimental.pallas import tpu as pltpu
from jax.experimental.pallas import tpu_sc as plsc

U32 = jnp.uint32
I32 = jnp.int32
F32 = jnp.float32
_SCALE15625 = np.float32((1.0 - 2.0**-21) / 15625.0)


def _c(v):
    return U32(v)


def _mul64c(lo, hi, c):
    c = _c(c)
    l0 = lo & _c(0xFFFF)
    l1 = lo >> _c(16)
    p0 = l0 * c
    p1 = l1 * c
    nlo = (p1 << _c(16)) + p0
    ch = (p1 + (p0 >> _c(16))) >> _c(16)
    h0 = hi & _c(0xFFFF)
    h1 = hi >> _c(16)
    nhi = ((h1 * c) << _c(16)) + h0 * c + ch
    return nlo, nhi


def _xor_shr(lo, hi, s):
    slo = (lo >> _c(s)) | (hi << _c(32 - s))
    shi = hi >> _c(s)
    return lo ^ slo, hi ^ shi


def _hash_i32(v):
    return plsc.bitcast(_hash_vec(plsc.bitcast(v, U32)), I32)


def _hash_vec(x):
    blo = x << _c(21)
    bhi = x >> _c(11)
    alo = ~x
    lo = alo + blo
    carry = jnp.where(lo < alo, _c(1), _c(0))
    hi = bhi + carry + _c(0xFFFFFFFF)
    lo, hi = _xor_shr(lo, hi, 24)
    lo, hi = _mul64c(lo, hi, 265)
    lo, hi = _xor_shr(lo, hi, 14)
    lo, hi = _mul64c(lo, hi, 21)
    lo, hi = _xor_shr(lo, hi, 28)
    b0m = lo << _c(31)
    nlo = lo ^ b0m
    c7 = (lo & b0m) >> _c(31)
    shi = (hi << _c(31)) | (lo >> _c(1))
    hi = hi + shi + c7
    lo = nlo
    r0 = lo & _c(63)
    qlo = (lo >> _c(6)) | (hi << _c(26))
    qhi = hi >> _c(6)
    w0 = qlo & _c(0xFFFF)
    w1 = qlo >> _c(16)
    w2 = qhi & _c(0xFFFF)
    w3 = qhi >> _c(16)
    s = w0 + w1 * _c(3036) + w2 * _c(14171) + w3 * _c(7531)
    s = s.astype(I32)
    q = (s.astype(F32) * _SCALE15625).astype(I32)
    r = s - q * I32(15625)
    t = r - I32(15625)
    r = t + ((t >> I32(31)) & I32(15625))
    return (r.astype(U32) << _c(6)) | r0


def _make_sc_call(n):
    info = plsc.get_sparse_core_info()
    nc, ns = info.num_cores, info.num_subcores
    nw = nc * ns
    per_w = n // nw
    groups = 4
    chunk = per_w // groups
    assert per_w * nw == n and chunk * groups == per_w and chunk % 16 == 0
    mesh = plsc.VectorSubcoreMesh(core_axis_name="c", subcore_axis_name="s")

    @functools.partial(
        pl.kernel,
        mesh=mesh,
        out_type=jax.ShapeDtypeStruct((n,), jnp.uint32),
        scratch_types=[pltpu.VMEM((per_w,), jnp.uint32)],
    )
    def sc_hash(x_hbm, out_hbm, buf):
        wid = lax.axis_index("s") * I32(nc) + lax.axis_index("c")
        base = pl.multiple_of(wid * I32(per_w), per_w)
        pltpu.sync_copy(x_hbm.at[pl.ds(base, per_w)], buf)

        @plsc.parallel_loop(I32(0), I32(per_w), step=I32(16), unroll=8)
        def _(i):
            off = pl.multiple_of(i, 16)
            buf[pl.ds(off, 16)] = _hash_vec(buf[pl.ds(off, 16)])

        pltpu.sync_copy(buf, out_hbm.at[pl.ds(base, per_w)])

    return sc_hash


@jax.jit
def kernel(raw_ids):
    shape = raw_ids.shape
    n = raw_ids.size
    x = raw_ids.astype(jnp.uint32).reshape(n)
    out = _make_sc_call(n)(x)
    return out.reshape(shape).astype(jnp.int64)


# speedup vs baseline: 1.2371x; 1.1501x over previous
import functools

import jax
import jax.numpy as jnp
import numpy as np
from jax import lax
from jax.experimental import pallas as pl
from jax.experimental.pallas import tpu as pltpu
from jax.experimental.pallas import tpu_sc as plsc

U32 = jnp.uint32
I32 = jnp.int32
F32 = jnp.float32
_SCALE15625 = np.float32((1.0 - 2.0**-21) / 15625.0)


def _c(v):
    return U32(v)


def _mul64c(lo, hi, c):
    c = _c(c)
    l0 = lo & _c(0xFFFF)
    l1 = lo >> _c(16)
    p0 = l0 * c
    p1 = l1 * c
    nlo = (p1 << _c(16)) + p0
    ch = (p1 + (p0 >> _c(16))) >> _c(16)
    h0 = hi & _c(0xFFFF)
    h1 = hi >> _c(16)
    nhi = ((h1 * c) << _c(16)) + h0 * c + ch
    return nlo, nhi


def _xor_shr(lo, hi, s):
    slo = (lo >> _c(s)) | (hi << _c(32 - s))
    shi = hi >> _c(s)
    return lo ^ slo, hi ^ shi


def _hash_vec(x):
    blo = x << _c(21)
    bhi = x >> _c(11)
    alo = ~x
    lo = alo + blo
    carry = jnp.where(lo < alo, _c(1), _c(0))
    hi = bhi + carry + _c(0xFFFFFFFF)
    lo, hi = _xor_shr(lo, hi, 24)
    lo, hi = _mul64c(lo, hi, 265)
    lo, hi = _xor_shr(lo, hi, 14)
    lo, hi = _mul64c(lo, hi, 21)
    lo, hi = _xor_shr(lo, hi, 28)
    b0m = lo << _c(31)
    nlo = lo ^ b0m
    c7 = (lo & b0m) >> _c(31)
    shi = (hi << _c(31)) | (lo >> _c(1))
    hi = hi + shi + c7
    lo = nlo
    r0 = lo & _c(63)
    qlo = (lo >> _c(6)) | (hi << _c(26))
    qhi = hi >> _c(6)
    w0 = qlo & _c(0xFFFF)
    w1 = qlo >> _c(16)
    w2 = qhi & _c(0xFFFF)
    w3 = qhi >> _c(16)
    s = w0 + w1 * _c(3036) + w2 * _c(14171) + w3 * _c(7531)
    s = s.astype(I32)
    q = (s.astype(F32) * _SCALE15625).astype(I32)
    r = s - q * I32(15625)
    t = r - I32(15625)
    r = t + ((t >> I32(31)) & I32(15625))
    return (r.astype(U32) << _c(6)) | r0


def _make_sc_call(rows, cols, sc_rows):
    info = plsc.get_sparse_core_info()
    nc, ns = info.num_cores, info.num_subcores
    nw = nc * ns
    rows_per_w = sc_rows // nw
    assert rows_per_w * nw == sc_rows
    row0 = rows - sc_rows
    n_full = cols // 16
    has_tail = cols % 16 != 0
    mesh = plsc.VectorSubcoreMesh(core_axis_name="c", subcore_axis_name="s")

    @functools.partial(
        pl.kernel,
        mesh=mesh,
        out_type=jax.ShapeDtypeStruct((sc_rows, cols), jnp.uint32),
        scratch_types=[pltpu.VMEM((rows_per_w, cols), jnp.uint32)],
    )
    def sc_hash(x_hbm, out_hbm, buf):
        wid = lax.axis_index("s") * I32(nc) + lax.axis_index("c")
        rb = wid * I32(rows_per_w)
        src = pl.multiple_of(I32(row0) + rb, rows_per_w)
        pltpu.sync_copy(x_hbm.at[pl.ds(src, rows_per_w)], buf)

        @plsc.parallel_loop(I32(0), I32(rows_per_w), step=I32(1), unroll=1)
        def _(r):
            for k in range(n_full - 1):
                buf[r, pl.ds(16 * k, 16)] = _hash_vec(
                    buf[r, pl.ds(16 * k, 16)])
            o = 16 * (n_full - 1)
            v_last = _hash_vec(buf[r, pl.ds(o, 16)])
            if has_tail:
                v_tail = _hash_vec(buf[r, pl.ds(cols - 16, 16)])
            buf[r, pl.ds(o, 16)] = v_last
            if has_tail:
                buf[r, pl.ds(cols - 16, 16)] = v_tail

        dst = pl.multiple_of(rb, rows_per_w)
        pltpu.sync_copy(buf, out_hbm.at[pl.ds(dst, rows_per_w)])

    return sc_hash


def _make_tc_call(rows, cols, tc_rows, blk_rows):
    assert tc_rows % blk_rows == 0

    def body(x_ref, o_ref):
        o_ref[...] = _hash_vec(x_ref[...])

    def imap(i):
        return (i, jnp.int32(0))

    return pl.pallas_call(
        body,
        grid=(tc_rows // blk_rows,),
        in_specs=[pl.BlockSpec((blk_rows, cols), imap)],
        out_specs=pl.BlockSpec((blk_rows, cols), imap),
        out_shape=jax.ShapeDtypeStruct((tc_rows, cols), jnp.uint32),
    )


@jax.jit
def kernel(raw_ids):
    rows, cols = raw_ids.shape
    sc_rows = rows // 4
    tc_rows = rows - sc_rows
    x = raw_ids.astype(jnp.uint32)
    sc_out = _make_sc_call(rows, cols, sc_rows)(x)
    tc_out = _make_tc_call(rows, cols, tc_rows, 1024)(x)
    out = jnp.concatenate([tc_out, sc_out], axis=0)
    return out.astype(jnp.int64)
